# baseline (speedup 1.0000x reference)
import functools

import jax
import jax.numpy as jnp
from jax import lax
from jax.experimental import pallas as pl
from jax.experimental.pallas import tpu as pltpu
from jax.experimental.pallas import tpu_sc as plsc

B = 1024
CTX = 50
D = 32
DP = 128
V = 100000

NC = 2
NS = 16
NW = NC * NS
B_PER_W = B // NW
ROWS_PER_W = B_PER_W * CTX
N_CHUNK = 2
B_PER_CHUNK = B_PER_W // N_CHUNK
ROWS_PER_CHUNK = ROWS_PER_W // N_CHUNK

_sc_mesh = plsc.VectorSubcoreMesh(core_axis_name="c", subcore_axis_name="s")


@functools.partial(
    pl.kernel,
    out_type=jax.ShapeDtypeStruct((B, D), jnp.float32),
    mesh=_sc_mesh,
    scratch_types=[
        pltpu.VMEM((ROWS_PER_CHUNK,), jnp.int32),
        pltpu.VMEM((ROWS_PER_CHUNK, DP), jnp.float32),
        pltpu.VMEM((B_PER_W, D), jnp.float32),
        pltpu.SemaphoreType.DMA,
    ],
    compiler_params=pltpu.CompilerParams(use_tc_tiling_on_sc=False),
)
def _pool_sc(idx_hbm, table_hbm, out_hbm, idx_v, rows_v, pooled_v, sem):
    wid = lax.axis_index("s") * NC + lax.axis_index("c")
    inv = jnp.float32(1.0 / CTX)

    def chunk_body(ck, carry0):
        base = wid * ROWS_PER_W + ck * ROWS_PER_CHUNK
        pltpu.sync_copy(idx_hbm.at[pl.ds(base, ROWS_PER_CHUNK)], idx_v)
        pltpu.async_copy(table_hbm.at[idx_v], rows_v, sem).wait()

        def body_b(b, carry):
            def body_c(c, acc):
                a0, a1 = acc
                r = b * CTX + c
                a0 = a0 + rows_v[r, pl.ds(0, 16)]
                a1 = a1 + rows_v[r, pl.ds(16, 16)]
                return (a0, a1)

            a0, a1 = lax.fori_loop(
                0, CTX, body_c,
                (jnp.zeros((16,), jnp.float32), jnp.zeros((16,), jnp.float32)),
            )
            bb = ck * B_PER_CHUNK + b
            pooled_v[bb, pl.ds(0, 16)] = a0 * inv
            pooled_v[bb, pl.ds(16, 16)] = a1 * inv
            return carry

        lax.fori_loop(0, B_PER_CHUNK, body_b, 0)
        return carry0

    lax.fori_loop(0, N_CHUNK, chunk_body, 0)
    pltpu.sync_copy(pooled_v, out_hbm.at[pl.ds(wid * B_PER_W, B_PER_W)])


VR = 2000
_N_VR = V // VR


def _pad_tc(x_ref, o_ref):
    o_ref[:, 0:D] = x_ref[...]


def _pad_table(emb_table):
    return pl.pallas_call(
        _pad_tc,
        grid=(_N_VR,),
        in_specs=[pl.BlockSpec((VR, D), lambda j: (j, 0))],
        out_specs=pl.BlockSpec((VR, DP), lambda j: (j, 0)),
        out_shape=jax.ShapeDtypeStruct((V, DP), jnp.float32),
    )(emb_table)


VB = 2048
_N_VB = (V + VB - 1) // VB


def _matmul_tc(w_ref, x_ref, b_ref, o_ref):
    dgn = (((0,), (0,)), ((), ()))
    o_ref[...] = jax.lax.dot_general(
        w_ref[...], x_ref[...], dgn, preferred_element_type=jnp.float32
    ) + jax.lax.dot_general(
        b_ref[...], jnp.ones((1, B), jnp.float32), dgn,
        preferred_element_type=jnp.float32,
    )


@jax.jit
def kernel(inputs, emb_table, dense_W, dense_b):
    idx = inputs.reshape(-1).astype(jnp.int32)
    table_p = _pad_table(emb_table)
    pooled = _pool_sc(idx, table_p)
    logits_t = pl.pallas_call(
        _matmul_tc,
        grid=(_N_VB,),
        in_specs=[
            pl.BlockSpec((D, VB), lambda j: (0, j)),
            pl.BlockSpec((D, B), lambda j: (0, 0)),
            pl.BlockSpec((1, VB), lambda j: (0, j)),
        ],
        out_specs=pl.BlockSpec((VB, B), lambda j: (j, 0)),
        out_shape=jax.ShapeDtypeStruct((V, B), jnp.float32),
    )(dense_W, pooled.T, dense_b[None, :])
    return logits_t.T


# speedup vs baseline: 1.1637x; 1.1637x over previous
import functools

import jax
import jax.numpy as jnp
from jax import lax
from jax.experimental import pallas as pl
from jax.experimental.pallas import tpu as pltpu
from jax.experimental.pallas import tpu_sc as plsc

B = 1024
CTX = 50
D = 32
DP = 128
V = 100000

NC = 2
NS = 16
NW = NC * NS
B_PER_W = B // NW
ROWS_PER_W = B_PER_W * CTX
N_CHUNK = 2
B_PER_CHUNK = B_PER_W // N_CHUNK
ROWS_PER_CHUNK = ROWS_PER_W // N_CHUNK

_sc_mesh = plsc.VectorSubcoreMesh(core_axis_name="c", subcore_axis_name="s")


@functools.partial(
    pl.kernel,
    out_type=jax.ShapeDtypeStruct((B, D), jnp.float32),
    mesh=_sc_mesh,
    scratch_types=[
        pltpu.VMEM((ROWS_PER_CHUNK,), jnp.int32),
        pltpu.VMEM((ROWS_PER_CHUNK, DP), jnp.float32),
        pltpu.VMEM((B_PER_W, D), jnp.float32),
        pltpu.SemaphoreType.DMA,
    ],
    compiler_params=pltpu.CompilerParams(use_tc_tiling_on_sc=False),
)
def _pool_sc(idx_hbm, table_hbm, out_hbm, idx_v, rows_v, pooled_v, sem):
    wid = lax.axis_index("s") * NC + lax.axis_index("c")
    inv = jnp.float32(1.0 / CTX)

    def chunk_body(ck, carry0):
        base = wid * ROWS_PER_W + ck * ROWS_PER_CHUNK
        pltpu.sync_copy(idx_hbm.at[pl.ds(base, ROWS_PER_CHUNK)], idx_v)
        pltpu.async_copy(table_hbm.at[idx_v], rows_v, sem).wait()

        def body_b(b, carry):
            def body_c(c, acc):
                a0, a1 = acc
                r = b * CTX + c
                a0 = a0 + rows_v[r, pl.ds(0, 16)]
                a1 = a1 + rows_v[r, pl.ds(16, 16)]
                return (a0, a1)

            a0, a1 = lax.fori_loop(
                0, CTX, body_c,
                (jnp.zeros((16,), jnp.float32), jnp.zeros((16,), jnp.float32)),
            )
            bb = ck * B_PER_CHUNK + b
            pooled_v[bb, pl.ds(0, 16)] = a0 * inv
            pooled_v[bb, pl.ds(16, 16)] = a1 * inv
            return carry

        lax.fori_loop(0, B_PER_CHUNK, body_b, 0)
        return carry0

    lax.fori_loop(0, N_CHUNK, chunk_body, 0)
    pltpu.sync_copy(pooled_v, out_hbm.at[pl.ds(wid * B_PER_W, B_PER_W)])


VB2 = 2048
_N_VB2 = (V + VB2 - 1) // VB2


def _tpad_tc(xt_ref, o_ref):
    eye = jnp.eye(D, dtype=jnp.float32)
    o_ref[:, 0:D] = jax.lax.dot_general(
        xt_ref[...], eye, (((0,), (0,)), ((), ())),
        preferred_element_type=jnp.float32,
    )


def _transpose_pad(table_t):
    return pl.pallas_call(
        _tpad_tc,
        grid=(_N_VB2,),
        in_specs=[pl.BlockSpec((D, VB2), lambda j: (0, j))],
        out_specs=pl.BlockSpec((VB2, DP), lambda j: (j, 0)),
        out_shape=jax.ShapeDtypeStruct((V, DP), jnp.float32),
    )(table_t)


VB = 2048
_N_VB = (V + VB - 1) // VB


def _matmul_tc(w_ref, x_ref, b_ref, o_ref):
    dgn = (((0,), (0,)), ((), ()))
    o_ref[...] = jax.lax.dot_general(
        w_ref[...], x_ref[...], dgn, preferred_element_type=jnp.float32
    ) + jax.lax.dot_general(
        b_ref[...], jnp.ones((1, B), jnp.float32), dgn,
        preferred_element_type=jnp.float32,
    )


@jax.jit
def kernel(inputs, emb_table, dense_W, dense_b):
    idx = inputs.reshape(-1).astype(jnp.int32)
    table_p = _transpose_pad(emb_table.T)
    pooled = _pool_sc(idx, table_p)
    logits_t = pl.pallas_call(
        _matmul_tc,
        grid=(_N_VB,),
        in_specs=[
            pl.BlockSpec((D, VB), lambda j: (0, j)),
            pl.BlockSpec((D, B), lambda j: (0, 0)),
            pl.BlockSpec((1, VB), lambda j: (0, j)),
        ],
        out_specs=pl.BlockSpec((VB, B), lambda j: (j, 0)),
        out_shape=jax.ShapeDtypeStruct((V, B), jnp.float32),
    )(dense_W, pooled.T, dense_b[None, :])
    return logits_t.T


# speedup vs baseline: 1.2869x; 1.1058x over previous
import functools

import jax
import jax.numpy as jnp
from jax import lax
from jax.experimental import pallas as pl
from jax.experimental.pallas import tpu as pltpu
from jax.experimental.pallas import tpu_sc as plsc

B = 1024
CTX = 50
D = 32
DP = 128
V = 100000

NC = 2
NS = 16
NW = NC * NS
B_PER_W = B // NW
ROWS_PER_W = B_PER_W * CTX
N_CHUNK = 2
B_PER_CHUNK = B_PER_W // N_CHUNK
ROWS_PER_CHUNK = ROWS_PER_W // N_CHUNK

_sc_mesh = plsc.VectorSubcoreMesh(core_axis_name="c", subcore_axis_name="s")


@functools.partial(
    pl.kernel,
    out_type=jax.ShapeDtypeStruct((B, D), jnp.float32),
    mesh=_sc_mesh,
    scratch_types=[
        pltpu.VMEM((ROWS_PER_CHUNK,), jnp.int32),
        pltpu.VMEM((ROWS_PER_CHUNK, DP), jnp.float32),
        pltpu.VMEM((B_PER_W, D), jnp.float32),
        pltpu.SemaphoreType.DMA,
    ],
    compiler_params=pltpu.CompilerParams(use_tc_tiling_on_sc=False),
)
def _pool_sc(idx_hbm, table_hbm, out_hbm, idx_v, rows_v, pooled_v, sem):
    wid = lax.axis_index("s") * NC + lax.axis_index("c")
    inv = jnp.float32(1.0 / CTX)

    def chunk_body(ck, carry0):
        base = wid * ROWS_PER_W + ck * ROWS_PER_CHUNK
        pltpu.sync_copy(idx_hbm.at[pl.ds(base, ROWS_PER_CHUNK)], idx_v)
        pltpu.async_copy(table_hbm.at[idx_v], rows_v, sem).wait()

        def body_b(b, carry):
            def body_c(c, acc):
                a0, a1 = acc
                r = b * CTX + c
                a0 = a0 + rows_v[r, pl.ds(0, 16)]
                a1 = a1 + rows_v[r, pl.ds(16, 16)]
                return (a0, a1)

            a0, a1 = lax.fori_loop(
                0, CTX, body_c,
                (jnp.zeros((16,), jnp.float32), jnp.zeros((16,), jnp.float32)),
            )
            bb = ck * B_PER_CHUNK + b
            pooled_v[bb, pl.ds(0, 16)] = a0 * inv
            pooled_v[bb, pl.ds(16, 16)] = a1 * inv
            return carry

        lax.fori_loop(0, B_PER_CHUNK, body_b, 0)
        return carry0

    lax.fori_loop(0, N_CHUNK, chunk_body, 0)
    pltpu.sync_copy(pooled_v, out_hbm.at[pl.ds(wid * B_PER_W, B_PER_W)])


VB2 = 8192
_N_VB2 = (V + VB2 - 1) // VB2


def _tpad_tc(xt_ref, o_ref):
    eye = jnp.eye(D, dtype=jnp.float32)
    o_ref[:, 0:D] = jax.lax.dot_general(
        xt_ref[...], eye, (((0,), (0,)), ((), ())),
        preferred_element_type=jnp.float32,
    )


def _transpose_pad(table_t):
    return pl.pallas_call(
        _tpad_tc,
        grid=(_N_VB2,),
        in_specs=[pl.BlockSpec((D, VB2), lambda j: (0, j))],
        out_specs=pl.BlockSpec((VB2, DP), lambda j: (j, 0)),
        out_shape=jax.ShapeDtypeStruct((V, DP), jnp.float32),
    )(table_t)


VB = 4096
_N_VB = (V + VB - 1) // VB


def _matmul_tc(w_ref, x_ref, b_ref, o_ref):
    dgn = (((0,), (0,)), ((), ()))
    o_ref[...] = jax.lax.dot_general(
        w_ref[...], x_ref[...], dgn, preferred_element_type=jnp.float32
    ) + jax.lax.dot_general(
        b_ref[...], jnp.ones((1, B), jnp.float32), dgn,
        preferred_element_type=jnp.float32,
    )


@jax.jit
def kernel(inputs, emb_table, dense_W, dense_b):
    idx = inputs.reshape(-1).astype(jnp.int32)
    table_p = _transpose_pad(emb_table.T)
    pooled = _pool_sc(idx, table_p)
    logits_t = pl.pallas_call(
        _matmul_tc,
        grid=(_N_VB,),
        in_specs=[
            pl.BlockSpec((D, VB), lambda j: (0, j)),
            pl.BlockSpec((D, B), lambda j: (0, 0)),
            pl.BlockSpec((1, VB), lambda j: (0, j)),
        ],
        out_specs=pl.BlockSpec((VB, B), lambda j: (j, 0)),
        out_shape=jax.ShapeDtypeStruct((V, B), jnp.float32),
    )(dense_W, pooled.T, dense_b[None, :])
    return logits_t.T
